# Initial kernel scaffold; baseline (speedup 1.0000x reference)
import jax
import jax.numpy as jnp
from jax.experimental import pallas as pl


def kernel(input_ids, position_ids, token_type_ids, word_emb, pos_emb, type_emb, ln_gamma, ln_beta):
    raise NotImplementedError("write your pallas kernel here")


# speedup vs baseline: 2.0233x; 2.0233x over previous
import functools

import jax
import jax.numpy as jnp
from jax import lax
from jax.experimental import pallas as pl
from jax.experimental.pallas import tpu as pltpu
from jax.experimental.pallas import tpu_sc as plsc

NC = 2
NS = 16
NW = NC * NS
L = 16
CH = 16
HALF = CH // 2


def _rsqrt_vec(v):
    seed = jnp.int32(0x5F3759DF) - (lax.bitcast_convert_type(v, jnp.int32) >> 1)
    y = lax.bitcast_convert_type(seed, jnp.float32)
    for _ in range(3):
        y = y * (1.5 - 0.5 * v * y * y)
    return y


def _emb_ln_call(ids3, pids3, tf3, word_emb, pos_emb, type_emb, gamma, beta):
    n_tok = ids3.shape[0]
    H = word_emb.shape[1]
    HJ = H // L
    tpw = n_tok // NW
    nch = tpw // CH
    npair = nch // 2

    mesh = plsc.VectorSubcoreMesh(
        core_axis_name="c", subcore_axis_name="s",
        num_cores=NC, num_subcores=NS)

    @functools.partial(
        pl.kernel,
        out_type=jax.ShapeDtypeStruct((n_tok, H), jnp.float32),
        mesh=mesh,
        scratch_types=[
            pltpu.VMEM((tpw,), jnp.int32),
            pltpu.VMEM((tpw,), jnp.int32),
            pltpu.VMEM((tpw,), jnp.float32),
            pltpu.VMEM((2, H), jnp.float32),
            pltpu.VMEM((H,), jnp.float32),
            pltpu.VMEM((H,), jnp.float32),
            pltpu.VMEM((CH, H), jnp.float32),
            pltpu.VMEM((CH, H), jnp.float32),
            pltpu.VMEM((CH, H // 2), jnp.int32),
            pltpu.VMEM((CH, H // 2), jnp.int32),
            pltpu.VMEM((CH, H), jnp.float32),
            pltpu.VMEM((CH, H), jnp.float32),
            pltpu.SemaphoreType.DMA,
            pltpu.SemaphoreType.DMA,
            pltpu.SemaphoreType.DMA,
            pltpu.SemaphoreType.DMA,
            pltpu.SemaphoreType.DMA,
        ],
        compiler_params=pltpu.CompilerParams(needs_layout_passes=False),
    )
    def run(ids_h, pids_h, tf_h, word_h, pos_h, typ_h, gamma_h, beta_h,
            out_h, idw, idp, tfv, typv, gvec, bvec,
            wb0, wb1, pb0, pb1, ob0, ob1, gs0, gs1, os0, os1, ssem):
        wid = lax.axis_index("s") * NC + lax.axis_index("c")
        base = wid * tpw
        d_ids = pltpu.async_copy(ids_h.at[pl.ds(base, tpw)], idw, ssem)
        d_pids = pltpu.async_copy(pids_h.at[pl.ds(base, tpw)], idp, ssem)
        d_tf = pltpu.async_copy(tf_h.at[pl.ds(base, tpw)], tfv, ssem)
        d_typ = pltpu.async_copy(typ_h, typv, ssem)
        d_g = pltpu.async_copy(gamma_h, gvec, ssem)
        d_b = pltpu.async_copy(beta_h, bvec, ssem)

        d_ids.wait()
        d_pids.wait()

        wbufs = (wb0, wb1)
        pbufs = (pb0, pb1)
        obufs = (ob0, ob1)
        gsems = (gs0, gs1)
        osems = (os0, os1)

        def issue_gather(c, b):
            pltpu.async_copy(word_h.at[idw.at[pl.ds(c * CH, CH)]],
                             wbufs[b], gsems[b])
            pltpu.async_copy(pos_h.at[idp.at[pl.ds(c * CH, CH)]],
                             pbufs[b], gsems[b])

        gdrain_w = tuple(
            pltpu.make_async_copy(out_h.at[pl.ds(0, CH)], wbufs[b], gsems[b])
            for b in range(2))
        gdrain_p = tuple(
            pltpu.make_async_copy(pos_h.at[pl.ds(0, CH)], pbufs[b], gsems[b])
            for b in range(2))
        odrain = tuple(
            pltpu.make_async_copy(obufs[b], out_h.at[pl.ds(0, CH)], osems[b])
            for b in range(2))

        def compute_chunk(c, b, drain_out):
            wbuf = wbufs[b]
            pbuf = pbufs[b]
            obuf = obufs[b]
            zero = jnp.zeros((L,), jnp.float32)

            @pl.when(drain_out)
            def _():
                odrain[b].wait()

            for h in range(CH // HALF):
                tmask = [plsc.load_gather(
                    tfv, [jnp.full((L,), c * CH + h * HALF + i, jnp.int32)])
                    > 0.5 for i in range(HALF)]

                def p1(j2, carry, h=h, tmask=tmask):
                    slo = pl.ds(j2 * 2 * L, L)
                    shi = pl.ds(j2 * 2 * L + L, L)
                    t0lo = typv[0, slo]
                    t0hi = typv[0, shi]
                    t1lo = typv[1, slo]
                    t1hi = typv[1, shi]
                    out = []
                    for i in range(HALF):
                        tok = h * HALF + i
                        pw = pbuf[tok, pl.ds(j2 * L, L)]
                        plo = lax.bitcast_convert_type(pw << 16, jnp.float32)
                        phi = lax.bitcast_convert_type(
                            pw & jnp.int32(-65536), jnp.float32)
                        xlo = wbuf[tok, slo] + plo + jnp.where(
                            tmask[i], t1lo, t0lo)
                        xhi = wbuf[tok, shi] + phi + jnp.where(
                            tmask[i], t1hi, t0hi)
                        wbuf[tok, slo] = xlo
                        wbuf[tok, shi] = xhi
                        out.append(carry[2 * i] + xlo + xhi)
                        out.append(carry[2 * i + 1]
                                   + xlo * xlo + xhi * xhi)
                    return tuple(out)

                accs = plsc.parallel_loop(0, HJ // 2,
                                          carry=(zero,) * (2 * HALF))(p1)

                mvs = []
                avs = []
                for i in range(HALF):
                    s = jnp.sum(accs[2 * i])
                    q = jnp.sum(accs[2 * i + 1])
                    mv = jnp.full((L,), s, jnp.float32) * (1.0 / H)
                    qv = jnp.full((L,), q, jnp.float32) * (1.0 / H)
                    mvs.append(mv)
                    avs.append(_rsqrt_vec(qv - mv * mv + 1e-12))

                @plsc.parallel_loop(0, HJ, unroll=2)
                def p2(j, h=h, mvs=mvs, avs=avs):
                    sl = pl.ds(j * L, L)
                    g = gvec[sl]
                    bta = bvec[sl]
                    for i in range(HALF):
                        tok = h * HALF + i
                        obuf[tok, sl] = ((wbuf[tok, sl] - mvs[i]) * avs[i]
                                         * g + bta)

        issue_gather(0, 0)
        d_tf.wait()
        d_typ.wait()
        d_g.wait()
        d_b.wait()

        @pl.loop(0, npair)
        def _pair(g):
            c0 = 2 * g
            gdrain_w[0].wait()
            gdrain_p[0].wait()
            issue_gather(c0 + 1, 1)
            compute_chunk(c0, 0, g > 0)
            pltpu.async_copy(obufs[0], out_h.at[pl.ds(base + c0 * CH, CH)],
                             osems[0])

            c1 = 2 * g + 1
            gdrain_w[1].wait()
            gdrain_p[1].wait()

            @pl.when(g < npair - 1)
            def _():
                issue_gather(c1 + 1, 0)
            compute_chunk(c1, 1, g > 0)
            pltpu.async_copy(obufs[1], out_h.at[pl.ds(base + c1 * CH, CH)],
                             osems[1])

        odrain[0].wait()
        odrain[1].wait()

    return run(ids3, pids3, tf3, word_emb, pos_emb, type_emb, gamma, beta)


def kernel(input_ids, position_ids, token_type_ids, word_emb, pos_emb,
           type_emb, ln_gamma, ln_beta):
    B, S = input_ids.shape
    n_tok = B * S
    tpw = n_tok // NW
    nch = tpw // CH
    ids3 = input_ids.astype(jnp.int32).reshape(-1)
    pids3 = position_ids.astype(jnp.int32).reshape(-1)
    tf3 = token_type_ids.astype(jnp.float32).reshape(-1)
    P, HP = pos_emb.shape
    posi = jax.lax.bitcast_convert_type(
        pos_emb.astype(jnp.bfloat16).reshape(P, HP // 32, 2, 16)
        .transpose(0, 1, 3, 2), jnp.int32).reshape(P, HP // 2)
    out = _emb_ln_call(ids3, pids3, tf3,
                       word_emb.astype(jnp.float32),
                       posi,
                       type_emb.astype(jnp.float32),
                       ln_gamma.astype(jnp.float32),
                       ln_beta.astype(jnp.float32))
    return out.reshape(B, S, -1)


# speedup vs baseline: 2.8076x; 1.3876x over previous
import functools

import jax
import jax.numpy as jnp
from jax import lax
from jax.experimental import pallas as pl
from jax.experimental.pallas import tpu as pltpu
from jax.experimental.pallas import tpu_sc as plsc

NC = 2
NS = 16
NW = NC * NS
L = 16
CH = 16
HALF = CH // 2


def _rsqrt_vec(v):
    seed = jnp.int32(0x5F3759DF) - (lax.bitcast_convert_type(v, jnp.int32) >> 1)
    y = lax.bitcast_convert_type(seed, jnp.float32)
    for _ in range(3):
        y = y * (1.5 - 0.5 * v * y * y)
    return y


def _emb_ln_call(ids3, pids3, tf3, word_emb, pos_emb, type_emb, gamma, beta):
    n_tok = ids3.shape[0]
    H = word_emb.shape[1]
    HJ = H // L
    tpw = n_tok // NW
    nch = tpw // CH
    npair = nch // 2

    mesh = plsc.VectorSubcoreMesh(
        core_axis_name="c", subcore_axis_name="s",
        num_cores=NC, num_subcores=NS)

    @functools.partial(
        pl.kernel,
        out_type=jax.ShapeDtypeStruct((n_tok, H), jnp.float32),
        mesh=mesh,
        scratch_types=[
            pltpu.VMEM((tpw,), jnp.int32),
            pltpu.VMEM((tpw,), jnp.int32),
            pltpu.VMEM((tpw,), jnp.float32),
            pltpu.VMEM((2, H), jnp.float32),
            pltpu.VMEM((H,), jnp.float32),
            pltpu.VMEM((H,), jnp.float32),
            pltpu.VMEM((CH, H), jnp.float32),
            pltpu.VMEM((CH, H), jnp.float32),
            pltpu.VMEM((CH, H), jnp.float32),
            pltpu.VMEM((CH, H), jnp.float32),
            pltpu.VMEM((CH, H), jnp.float32),
            pltpu.VMEM((CH, H), jnp.float32),
            pltpu.SemaphoreType.DMA,
            pltpu.SemaphoreType.DMA,
            pltpu.SemaphoreType.DMA,
            pltpu.SemaphoreType.DMA,
            pltpu.SemaphoreType.DMA,
        ],
        compiler_params=pltpu.CompilerParams(needs_layout_passes=False),
    )
    def run(ids_h, pids_h, tf_h, word_h, pos_h, typ_h, gamma_h, beta_h,
            out_h, idw, idp, tfv, typv, gvec, bvec,
            wb0, wb1, pb0, pb1, ob0, ob1, gs0, gs1, os0, os1, ssem):
        wid = lax.axis_index("s") * NC + lax.axis_index("c")
        base = wid * tpw
        d_ids = pltpu.async_copy(ids_h.at[pl.ds(base, tpw)], idw, ssem)
        d_pids = pltpu.async_copy(pids_h.at[pl.ds(base, tpw)], idp, ssem)
        d_tf = pltpu.async_copy(tf_h.at[pl.ds(base, tpw)], tfv, ssem)
        d_typ = pltpu.async_copy(typ_h, typv, ssem)
        d_g = pltpu.async_copy(gamma_h, gvec, ssem)
        d_b = pltpu.async_copy(beta_h, bvec, ssem)

        d_ids.wait()
        d_pids.wait()

        wbufs = (wb0, wb1)
        pbufs = (pb0, pb1)
        obufs = (ob0, ob1)
        gsems = (gs0, gs1)
        osems = (os0, os1)

        def issue_gather(c, b):
            pltpu.async_copy(word_h.at[idw.at[pl.ds(c * CH, CH)]],
                             wbufs[b], gsems[b])
            pltpu.async_copy(pos_h.at[idp.at[pl.ds(c * CH, CH)]],
                             pbufs[b], gsems[b])

        gdrain_w = tuple(
            pltpu.make_async_copy(out_h.at[pl.ds(0, CH)], wbufs[b], gsems[b])
            for b in range(2))
        gdrain_p = tuple(
            pltpu.make_async_copy(pos_h.at[pl.ds(0, CH)], pbufs[b], gsems[b])
            for b in range(2))
        odrain = tuple(
            pltpu.make_async_copy(obufs[b], out_h.at[pl.ds(0, CH)], osems[b])
            for b in range(2))

        def compute_chunk(c, b, drain_out):
            wbuf = wbufs[b]
            pbuf = pbufs[b]
            obuf = obufs[b]
            zero = jnp.zeros((L,), jnp.float32)

            @pl.when(drain_out)
            def _():
                odrain[b].wait()

            for h in range(CH // HALF):
                tmask = [plsc.load_gather(
                    tfv, [jnp.full((L,), c * CH + h * HALF + i, jnp.int32)])
                    > 0.5 for i in range(HALF)]

                def p1(j, carry, h=h, tmask=tmask):
                    sl = pl.ds(j * L, L)
                    t0 = typv[0, sl]
                    t1 = typv[1, sl]
                    out = []
                    for i in range(HALF):
                        tok = h * HALF + i
                        x = (wbuf[tok, sl] + pbuf[tok, sl]
                             + jnp.where(tmask[i], t1, t0))
                        wbuf[tok, sl] = x
                        out.append(carry[2 * i] + x)
                        out.append(carry[2 * i + 1] + x * x)
                    return tuple(out)

                accs = plsc.parallel_loop(0, HJ, unroll=2,
                                          carry=(zero,) * (2 * HALF))(p1)

                mvs = []
                avs = []
                for i in range(HALF):
                    s = jnp.sum(accs[2 * i])
                    q = jnp.sum(accs[2 * i + 1])
                    mv = jnp.full((L,), s, jnp.float32) * (1.0 / H)
                    qv = jnp.full((L,), q, jnp.float32) * (1.0 / H)
                    mvs.append(mv)
                    avs.append(_rsqrt_vec(qv - mv * mv + 1e-12))

                @plsc.parallel_loop(0, HJ, unroll=2)
                def p2(j, h=h, mvs=mvs, avs=avs):
                    sl = pl.ds(j * L, L)
                    g = gvec[sl]
                    bta = bvec[sl]
                    for i in range(HALF):
                        tok = h * HALF + i
                        obuf[tok, sl] = ((wbuf[tok, sl] - mvs[i]) * avs[i]
                                         * g + bta)

        issue_gather(0, 0)
        d_tf.wait()
        d_typ.wait()
        d_g.wait()
        d_b.wait()

        @pl.loop(0, npair)
        def _pair(g):
            c0 = 2 * g
            gdrain_w[0].wait()
            gdrain_p[0].wait()
            issue_gather(c0 + 1, 1)
            compute_chunk(c0, 0, g > 0)
            pltpu.async_copy(obufs[0], out_h.at[pl.ds(base + c0 * CH, CH)],
                             osems[0])

            c1 = 2 * g + 1
            gdrain_w[1].wait()
            gdrain_p[1].wait()

            @pl.when(g < npair - 1)
            def _():
                issue_gather(c1 + 1, 0)
            compute_chunk(c1, 1, g > 0)
            pltpu.async_copy(obufs[1], out_h.at[pl.ds(base + c1 * CH, CH)],
                             osems[1])

        odrain[0].wait()
        odrain[1].wait()

    return run(ids3, pids3, tf3, word_emb, pos_emb, type_emb, gamma, beta)


def kernel(input_ids, position_ids, token_type_ids, word_emb, pos_emb,
           type_emb, ln_gamma, ln_beta):
    B, S = input_ids.shape
    n_tok = B * S
    tpw = n_tok // NW
    nch = tpw // CH
    ids3 = input_ids.astype(jnp.int32).reshape(-1)
    pids3 = position_ids.astype(jnp.int32).reshape(-1)
    tf3 = token_type_ids.astype(jnp.float32).reshape(-1)
    out = _emb_ln_call(ids3, pids3, tf3,
                       word_emb.astype(jnp.float32),
                       pos_emb.astype(jnp.float32),
                       type_emb.astype(jnp.float32),
                       ln_gamma.astype(jnp.float32),
                       ln_beta.astype(jnp.float32))
    return out.reshape(B, S, -1)
